# Initial kernel scaffold; baseline (speedup 1.0000x reference)
import jax
import jax.numpy as jnp
from jax.experimental import pallas as pl


def kernel(xyz_src, xyz_dst, feat_src, feat_dst, W0, b0, gamma0, beta0, W1, b1, gamma1, beta1):
    raise NotImplementedError("write your pallas kernel here")


# speedup vs baseline: 12.4236x; 12.4236x over previous
import functools

import jax
import jax.numpy as jnp
from jax.experimental import pallas as pl
from jax.experimental.pallas import tpu as pltpu
from jax.experimental.pallas import tpu_sc as plsc

B, N, M = 4, 4096, 1024
C = 256
IN_C = 2 * C
EPS_BN = 1e-5

TN_NN = 256
TN_MM = 512
NW = 32
GW = 128


def _nn_body(src_ref, dstT_ref, idx_ref, w_ref):
    b = pl.program_id(0)
    s = src_ref[0]
    t = dstT_ref[0]
    dx = s[:, 0:1] - t[0:1, :]
    dy = s[:, 1:2] - t[1:2, :]
    dz = s[:, 2:3] - t[2:3, :]
    d2 = dx * dx + dy * dy + dz * dz
    iota = jax.lax.broadcasted_iota(jnp.int32, d2.shape, 1)
    vals, idxs = [], []
    for _ in range(3):
        vmin = jnp.min(d2, axis=1, keepdims=True)
        imin = jnp.min(jnp.where(d2 == vmin, iota, M), axis=1, keepdims=True)
        vals.append(vmin)
        idxs.append(imin)
        d2 = jnp.where(iota == imin, jnp.float32(jnp.inf), d2)
    v3 = jnp.concatenate(vals, axis=1)
    i3 = jnp.concatenate(idxs, axis=1)
    d3 = jnp.sqrt(v3) + 1e-8
    w = 1.0 / d3
    w = w / jnp.sum(w, axis=1, keepdims=True)
    idx_ref[0] = i3 + b * M
    w_ref[0] = w


def _three_nn(xyz_src, xyz_dstT):
    return pl.pallas_call(
        _nn_body,
        grid=(B, N // TN_NN),
        in_specs=[
            pl.BlockSpec((1, TN_NN, 3), lambda b, i: (b, i, 0)),
            pl.BlockSpec((1, 3, M), lambda b, i: (b, 0, 0)),
        ],
        out_specs=[
            pl.BlockSpec((1, TN_NN, 3), lambda b, i: (b, i, 0)),
            pl.BlockSpec((1, TN_NN, 3), lambda b, i: (b, i, 0)),
        ],
        out_shape=[
            jax.ShapeDtypeStruct((B, N, 3), jnp.int32),
            jax.ShapeDtypeStruct((B, N, 3), jnp.float32),
        ],
    )(xyz_src, xyz_dstT)


def _sc_gather(table, gidx):
    NI = gidx.shape[0]
    per_w = NI // NW
    nch = per_w // GW
    mesh = plsc.VectorSubcoreMesh(core_axis_name="c", subcore_axis_name="s")

    @functools.partial(
        pl.kernel,
        mesh=mesh,
        out_type=jax.ShapeDtypeStruct((NI, C), jnp.float32),
        scratch_types=[
            pltpu.VMEM((GW,), jnp.int32),
            pltpu.VMEM((GW, C), jnp.float32),
            pltpu.SemaphoreType.DMA,
        ],
    )
    def k(table_hbm, idx_hbm, out_hbm, idx_v, rows_v, sem):
        wid = jax.lax.axis_index("s") * 2 + jax.lax.axis_index("c")
        base = wid * per_w

        @pl.loop(0, nch)
        def _(ci):
            off = base + ci * GW
            pltpu.sync_copy(idx_hbm.at[pl.ds(off, GW)], idx_v)
            pltpu.async_copy(table_hbm.at[idx_v], rows_v, sem).wait()
            pltpu.sync_copy(rows_v, out_hbm.at[pl.ds(off, GW)])

    return k(table, gidx)


def _l1_body(g_ref, w_ref, fsrc_ref, w0t_ref, b0_ref, y_ref, ps_ref, pss_ref):
    w = w_ref[...]
    interp = (g_ref[0] * w[:, 0:1] + g_ref[1] * w[:, 1:2]
              + g_ref[2] * w[:, 2:3])
    x = jnp.concatenate([interp, fsrc_ref[...]], axis=1)
    y = jnp.dot(x, w0t_ref[...], preferred_element_type=jnp.float32)
    y = y + b0_ref[...]
    y_ref[...] = y
    ps_ref[0, 0, :] = jnp.sum(y, axis=0)
    pss_ref[0, 0, :] = jnp.sum(y * y, axis=0)


def _layer1(gathered, wflat, fsrcT, W0T, b0row):
    steps = (B * N) // TN_MM
    return pl.pallas_call(
        _l1_body,
        grid=(steps,),
        in_specs=[
            pl.BlockSpec((3, TN_MM, C), lambda i: (0, i, 0)),
            pl.BlockSpec((TN_MM, 3), lambda i: (i, 0)),
            pl.BlockSpec((TN_MM, C), lambda i: (i, 0)),
            pl.BlockSpec((IN_C, C), lambda i: (0, 0)),
            pl.BlockSpec((1, C), lambda i: (0, 0)),
        ],
        out_specs=[
            pl.BlockSpec((TN_MM, C), lambda i: (i, 0)),
            pl.BlockSpec((1, 1, C), lambda i: (i, 0, 0)),
            pl.BlockSpec((1, 1, C), lambda i: (i, 0, 0)),
        ],
        out_shape=[
            jax.ShapeDtypeStruct((B * N, C), jnp.float32),
            jax.ShapeDtypeStruct((steps, 1, C), jnp.float32),
            jax.ShapeDtypeStruct((steps, 1, C), jnp.float32),
        ],
    )(gathered, wflat, fsrcT, W0T, b0row)


def _l2_body(y0_ref, sc_ref, sh_ref, w1t_ref, b1_ref, y_ref, ps_ref, pss_ref):
    h = jnp.maximum(y0_ref[...] * sc_ref[...] + sh_ref[...], 0.0)
    y = jnp.dot(h, w1t_ref[...], preferred_element_type=jnp.float32)
    y = y + b1_ref[...]
    y_ref[...] = y
    ps_ref[0, 0, :] = jnp.sum(y, axis=0)
    pss_ref[0, 0, :] = jnp.sum(y * y, axis=0)


def _layer2(y0, sc0, sh0, W1T, b1row):
    steps = (B * N) // TN_MM
    return pl.pallas_call(
        _l2_body,
        grid=(steps,),
        in_specs=[
            pl.BlockSpec((TN_MM, C), lambda i: (i, 0)),
            pl.BlockSpec((1, C), lambda i: (0, 0)),
            pl.BlockSpec((1, C), lambda i: (0, 0)),
            pl.BlockSpec((C, C), lambda i: (0, 0)),
            pl.BlockSpec((1, C), lambda i: (0, 0)),
        ],
        out_specs=[
            pl.BlockSpec((TN_MM, C), lambda i: (i, 0)),
            pl.BlockSpec((1, 1, C), lambda i: (i, 0, 0)),
            pl.BlockSpec((1, 1, C), lambda i: (i, 0, 0)),
        ],
        out_shape=[
            jax.ShapeDtypeStruct((B * N, C), jnp.float32),
            jax.ShapeDtypeStruct((steps, 1, C), jnp.float32),
            jax.ShapeDtypeStruct((steps, 1, C), jnp.float32),
        ],
    )(y0, sc0, sh0, W1T, b1row)


def _out_body(y1_ref, sc_ref, sh_ref, o_ref):
    h = jnp.maximum(y1_ref[0] * sc_ref[...] + sh_ref[...], 0.0)
    o_ref[0] = h.T


def _finalize(y1b, sc1, sh1):
    return pl.pallas_call(
        _out_body,
        grid=(B, N // TN_MM),
        in_specs=[
            pl.BlockSpec((1, TN_MM, C), lambda b, i: (b, i, 0)),
            pl.BlockSpec((1, C), lambda b, i: (0, 0)),
            pl.BlockSpec((1, C), lambda b, i: (0, 0)),
        ],
        out_specs=pl.BlockSpec((1, C, TN_MM), lambda b, i: (b, 0, i)),
        out_shape=jax.ShapeDtypeStruct((B, C, N), jnp.float32),
    )(y1b, sc1, sh1)


def kernel(xyz_src, xyz_dst, feat_src, feat_dst,
           W0, b0, gamma0, beta0, W1, b1, gamma1, beta1):
    xyz_dstT = jnp.transpose(xyz_dst, (0, 2, 1))
    idx, w = _three_nn(xyz_src, xyz_dstT)

    gidx = jnp.transpose(idx, (2, 0, 1)).reshape(3 * B * N)
    table = jnp.transpose(feat_dst, (0, 2, 1)).reshape(B * M, C)
    gathered = _sc_gather(table, gidx).reshape(3, B * N, C)

    fsrcT = jnp.transpose(feat_src, (0, 2, 1)).reshape(B * N, C)
    wflat = w.reshape(B * N, 3)
    y0, ps0, pss0 = _layer1(gathered, wflat, fsrcT,
                            W0.T, b0.reshape(1, C))

    n = jnp.float32(B * N)
    mu0 = jnp.sum(ps0, axis=0) / n
    var0 = jnp.sum(pss0, axis=0) / n - mu0 * mu0
    sc0 = gamma0 / jnp.sqrt(var0 + EPS_BN)
    sh0 = beta0 - mu0 * sc0

    y1, ps1, pss1 = _layer2(y0, sc0.reshape(1, C), sh0.reshape(1, C),
                            W1.T, b1.reshape(1, C))
    mu1 = jnp.sum(ps1, axis=0) / n
    var1 = jnp.sum(pss1, axis=0) / n - mu1 * mu1
    sc1 = gamma1 / jnp.sqrt(var1 + EPS_BN)
    sh1 = beta1 - mu1 * sc1

    return _finalize(y1.reshape(B, N, C),
                     sc1.reshape(1, C), sh1.reshape(1, C))
